# Initial kernel scaffold; baseline (speedup 1.0000x reference)
import jax
import jax.numpy as jnp
from jax.experimental import pallas as pl


def kernel(x, edge_index, edge_weight, W, b):
    raise NotImplementedError("write your pallas kernel here")


# speedup vs baseline: 15.1134x; 15.1134x over previous
import functools

import jax
import jax.numpy as jnp
from jax import lax
from jax.experimental import pallas as pl
from jax.experimental.pallas import tpu as pltpu
from jax.experimental.pallas import tpu_sc as plsc

NC = 2
NS = 16
NW = NC * NS
L = 16

BATCH = 128
FG = 128 // L

_MESH = plsc.VectorSubcoreMesh(core_axis_name="c", subcore_axis_name="s")


def _sc_deg(n_pad, nb):
    rows_per_tile = n_pad // NS

    @functools.partial(
        pl.kernel,
        out_type=[
            jax.ShapeDtypeStruct((n_pad,), jnp.float32),
            jax.ShapeDtypeStruct((n_pad,), jnp.float32),
        ],
        mesh=_MESH,
        compiler_params=pltpu.CompilerParams(needs_layout_passes=False),
        scratch_types=[
            pltpu.VMEM((nb, BATCH), jnp.int32),
            pltpu.VMEM((nb, BATCH), jnp.float32),
            pltpu.VMEM((rows_per_tile,), jnp.float32),
            pltpu.VMEM_SHARED((n_pad,), jnp.float32),
        ],
    )
    def deg_sc(col_g, w_g, out0, out1, colbuf, wbuf, zbuf, deg_sh):
        cid = lax.axis_index("c")
        sid = lax.axis_index("s")
        wid = sid * NC + cid
        row_base = sid * rows_per_tile
        zv = jnp.zeros((L,), jnp.float32)

        def zbody(t, c):
            zbuf[pl.ds(t * L, L)] = zv
            return c
        lax.fori_loop(0, rows_per_tile // L, zbody, 0)
        pltpu.sync_copy(zbuf, deg_sh.at[pl.ds(row_base, rows_per_tile)])
        plsc.subcore_barrier()

        pltpu.sync_copy(col_g.at[wid], colbuf)
        pltpu.sync_copy(w_g.at[wid], wbuf)

        def dbody(j, c):
            pltpu.sync_copy(wbuf.at[j], deg_sh.at[colbuf.at[j]], add=True)
            return c
        lax.fori_loop(0, nb, dbody, 0)
        plsc.subcore_barrier()

        src = deg_sh.at[pl.ds(row_base, rows_per_tile)]

        @pl.when(cid == 0)
        def _():
            pltpu.sync_copy(src, out0.at[pl.ds(row_base, rows_per_tile)])

        @pl.when(cid == 1)
        def _():
            pltpu.sync_copy(src, out1.at[pl.ds(row_base, rows_per_tile)])

    return deg_sc


def _sc_main(n_pad, nb, d, sb):
    rows_per_tile = n_pad // NS
    fgroups = d // L
    n_stages = nb // sb

    @functools.partial(
        pl.kernel,
        out_type=[
            jax.ShapeDtypeStruct((n_pad, d), jnp.float32),
            jax.ShapeDtypeStruct((n_pad, d), jnp.float32),
        ],
        mesh=_MESH,
        compiler_params=pltpu.CompilerParams(needs_layout_passes=False),
        scratch_types=[
            pltpu.VMEM((sb, BATCH), jnp.int32),
            pltpu.VMEM((sb, BATCH), jnp.int32),
            pltpu.VMEM((sb, BATCH), jnp.float32),
            pltpu.VMEM((BATCH, d), jnp.float32),
            pltpu.VMEM((n_pad,), jnp.float32),
            pltpu.VMEM((BATCH,), jnp.float32),
            pltpu.VMEM_SHARED((n_pad, d), jnp.float32),
            pltpu.SemaphoreType.DMA,
        ],
    )
    def gcn_sc(row_g, col_g, w_g, h_hbm, dinv_hbm, out0, out1,
               rowbuf, colbuf, wbuf, rows, dinv_loc, normbuf, acc, sem):
        cid = lax.axis_index("c")
        sid = lax.axis_index("s")
        wid = sid * NC + cid
        row_base = sid * rows_per_tile
        zv = jnp.zeros((L,), jnp.float32)

        def zbody(e, c):
            for f in range(fgroups):
                rows[e, pl.ds(f * L, L)] = zv
            return c
        lax.fori_loop(0, BATCH, zbody, 0)
        for k in range(rows_per_tile // BATCH):
            base = row_base + k * BATCH
            pltpu.sync_copy(rows, acc.at[pl.ds(base, BATCH)])
        plsc.subcore_barrier()

        pltpu.sync_copy(dinv_hbm, dinv_loc)

        def stage_body(st, c):
            chunk = wid * n_stages + st
            pltpu.sync_copy(row_g.at[chunk], rowbuf)
            pltpu.sync_copy(col_g.at[chunk], colbuf)
            pltpu.sync_copy(w_g.at[chunk], wbuf)

            def batch_body(j, c2):
                pltpu.async_copy(h_hbm.at[rowbuf.at[j]], rows, sem).wait()
                for g in range(FG):
                    sl = pl.ds(g * L, L)
                    rv = rowbuf[j, sl]
                    cv = colbuf[j, sl]
                    wv = wbuf[j, sl]
                    dr = plsc.load_gather(dinv_loc, [rv])
                    dc = plsc.load_gather(dinv_loc, [cv])
                    normbuf[sl] = dr * wv * dc

                def ebody(e, c3):
                    ei = jnp.broadcast_to(e, (L,)).astype(jnp.int32)
                    s = plsc.load_gather(normbuf, [ei])
                    for f in range(fgroups):
                        fs = pl.ds(f * L, L)
                        rows[e, fs] = rows[e, fs] * s
                    return c3
                lax.fori_loop(0, BATCH, ebody, 0)
                pltpu.sync_copy(rows, acc.at[colbuf.at[j]], add=True)
                return c2
            lax.fori_loop(0, sb, batch_body, c)
            return c
        lax.fori_loop(0, n_stages, stage_body, 0)
        plsc.subcore_barrier()

        src = acc.at[pl.ds(row_base, rows_per_tile)]

        @pl.when(cid == 0)
        def _():
            pltpu.sync_copy(src, out0.at[pl.ds(row_base, rows_per_tile)])

        @pl.when(cid == 1)
        def _():
            pltpu.sync_copy(src, out1.at[pl.ds(row_base, rows_per_tile)])

    return gcn_sc


def _tc_matmul_dinv(x, W, deg0, deg1, n_pad):
    n, d_in = x.shape
    d_out = W.shape[1]
    blk = 400
    rows_dinv = n_pad // 128

    def body(x_ref, w_ref, d0_ref, d1_ref, h_ref, dinv_ref):
        h_ref[...] = jnp.dot(x_ref[...], w_ref[...],
                             preferred_element_type=jnp.float32)
        deg = d0_ref[...] + d1_ref[...]
        dinv_ref[...] = jnp.where(
            deg > 0, lax.rsqrt(jnp.maximum(deg, 1e-12)), 0.0)

    h, dinv = pl.pallas_call(
        body,
        grid=(n // blk,),
        in_specs=[pl.BlockSpec((blk, d_in), lambda i: (i, 0)),
                  pl.BlockSpec((d_in, d_out), lambda i: (0, 0)),
                  pl.BlockSpec((rows_dinv, 128), lambda i: (0, 0)),
                  pl.BlockSpec((rows_dinv, 128), lambda i: (0, 0))],
        out_specs=[pl.BlockSpec((blk, d_out), lambda i: (i, 0)),
                   pl.BlockSpec((rows_dinv, 128), lambda i: (0, 0))],
        out_shape=[jax.ShapeDtypeStruct((n, d_out), jnp.float32),
                   jax.ShapeDtypeStruct((rows_dinv, 128), jnp.float32)],
    )(x, W, deg0.reshape(rows_dinv, 128), deg1.reshape(rows_dinv, 128))
    return h, dinv.reshape(n_pad)


def _tc_combine(p0, p1, b2d, n, d):
    blk = 80
    return pl.pallas_call(
        lambda a_ref, b_ref, c_ref, o_ref: o_ref.__setitem__(
            ..., a_ref[...] + b_ref[...] + c_ref[...]),
        grid=(n // blk,),
        in_specs=[pl.BlockSpec((blk, d), lambda i: (i, 0)),
                  pl.BlockSpec((blk, d), lambda i: (i, 0)),
                  pl.BlockSpec((1, d), lambda i: (0, 0))],
        out_specs=pl.BlockSpec((blk, d), lambda i: (i, 0)),
        out_shape=jax.ShapeDtypeStruct((n, d), jnp.float32),
    )(p0, p1, b2d)


def kernel(x, edge_index, edge_weight, W, b):
    n, d_in = x.shape
    d = W.shape[1]
    e = edge_index.shape[1]

    e_full = e + n
    per_tile_edges = -(-e_full // (NW * BATCH)) * BATCH
    e_pad = per_tile_edges * NW
    nb = per_tile_edges // BATCH
    pad = e_pad - e_full

    idx_dtype = edge_index.dtype
    loop_idx = jnp.arange(n, dtype=idx_dtype)
    zpad_i = jnp.zeros((pad,), dtype=idx_dtype)
    row_full = jnp.concatenate([edge_index[0], loop_idx, zpad_i])
    col_full = jnp.concatenate([edge_index[1], loop_idx, zpad_i])
    w_full = jnp.concatenate([edge_weight, jnp.ones((n,), jnp.float32),
                              jnp.zeros((pad,), jnp.float32)])
    row_g = row_full.reshape(NW, nb, BATCH).astype(jnp.int32)
    col_g = col_full.reshape(NW, nb, BATCH).astype(jnp.int32)
    w_g = w_full.reshape(NW, nb, BATCH)

    n_pad = -(-n // (NS * BATCH)) * (NS * BATCH)

    sb = next(s for s in (9, 3, 1) if nb % s == 0)
    n_stages = nb // sb
    row_s = row_g.reshape(NW * n_stages, sb, BATCH)
    col_s = col_g.reshape(NW * n_stages, sb, BATCH)
    w_s = w_g.reshape(NW * n_stages, sb, BATCH)
    deg0, deg1 = _sc_deg(n_pad, nb)(col_g, w_g)
    h, dinv = _tc_matmul_dinv(x, W, deg0, deg1, n_pad)
    p0, p1 = _sc_main(n_pad, nb, d, sb)(row_s, col_s, w_s, h, dinv)
    out = _tc_combine(p0, p1, b.reshape(1, d), n, d)
    return out


# speedup vs baseline: 18.9984x; 1.2571x over previous
import functools

import jax
import jax.numpy as jnp
from jax import lax
from jax.experimental import pallas as pl
from jax.experimental.pallas import tpu as pltpu
from jax.experimental.pallas import tpu_sc as plsc

NC = 2
NS = 16
NW = NC * NS
L = 16

BATCH = 128
FG = 128 // L

_MESH = plsc.VectorSubcoreMesh(core_axis_name="c", subcore_axis_name="s")


def _sc_deg(n_pad, nb):
    rows_per_tile = n_pad // NS

    @functools.partial(
        pl.kernel,
        out_type=[
            jax.ShapeDtypeStruct((n_pad,), jnp.float32),
            jax.ShapeDtypeStruct((n_pad,), jnp.float32),
        ],
        mesh=_MESH,
        compiler_params=pltpu.CompilerParams(needs_layout_passes=False),
        scratch_types=[
            pltpu.VMEM((nb, BATCH), jnp.int32),
            pltpu.VMEM((nb, BATCH), jnp.float32),
            pltpu.VMEM((-(-rows_per_tile // L) * L,), jnp.float32),
            pltpu.VMEM_SHARED((n_pad,), jnp.float32),
        ],
    )
    def deg_sc(col_g, w_g, out0, out1, colbuf, wbuf, zbuf, deg_sh):
        cid = lax.axis_index("c")
        sid = lax.axis_index("s")
        wid = sid * NC + cid
        row_base = sid * rows_per_tile
        zv = jnp.zeros((L,), jnp.float32)

        def zbody(t, c):
            zbuf[pl.ds(t * L, L)] = zv
            return c
        lax.fori_loop(0, -(-rows_per_tile // L), zbody, 0)
        pltpu.sync_copy(zbuf.at[pl.ds(0, rows_per_tile)],
                        deg_sh.at[pl.ds(row_base, rows_per_tile)])
        plsc.subcore_barrier()

        pltpu.sync_copy(col_g.at[wid], colbuf)
        pltpu.sync_copy(w_g.at[wid], wbuf)

        def dbody(j, c):
            pltpu.sync_copy(wbuf.at[j], deg_sh.at[colbuf.at[j]], add=True)
            return c
        lax.fori_loop(0, nb, dbody, 0)
        plsc.subcore_barrier()

        src = deg_sh.at[pl.ds(row_base, rows_per_tile)]

        @pl.when(cid == 0)
        def _():
            pltpu.sync_copy(src, out0.at[pl.ds(row_base, rows_per_tile)])

        @pl.when(cid == 1)
        def _():
            pltpu.sync_copy(src, out1.at[pl.ds(row_base, rows_per_tile)])

    return deg_sc


def _sc_main(n_pad, n_vec, nb, d, sb):
    rows_per_tile = n_pad // NS
    fgroups = d // L
    n_stages = nb // sb

    @functools.partial(
        pl.kernel,
        out_type=[
            jax.ShapeDtypeStruct((n_pad, d), jnp.float32),
            jax.ShapeDtypeStruct((n_pad, d), jnp.float32),
        ],
        mesh=_MESH,
        compiler_params=pltpu.CompilerParams(needs_layout_passes=False),
        scratch_types=[
            pltpu.VMEM((sb, BATCH), jnp.int32),
            pltpu.VMEM((sb, BATCH), jnp.int32),
            pltpu.VMEM((sb, BATCH), jnp.float32),
            pltpu.VMEM((BATCH, d), jnp.float32),
            pltpu.VMEM((BATCH, d), jnp.float32),
            pltpu.VMEM((n_vec,), jnp.float32),
            pltpu.VMEM((BATCH,), jnp.float32),
            pltpu.VMEM_SHARED((n_pad, d), jnp.float32),
            pltpu.SemaphoreType.DMA,
            pltpu.SemaphoreType.DMA,
            pltpu.SemaphoreType.DMA,
            pltpu.SemaphoreType.DMA,
        ],
    )
    def gcn_sc(row_g, col_g, w_g, h_hbm, dinv_hbm, out0, out1,
               rowbuf, colbuf, wbuf, rows_a, rows_b, dinv_loc, normbuf, acc,
               gsem_a, gsem_b, ssem_a, ssem_b):
        rows = rows_a
        cid = lax.axis_index("c")
        sid = lax.axis_index("s")
        wid = sid * NC + cid
        row_base = sid * rows_per_tile
        zv = jnp.zeros((L,), jnp.float32)

        def zbody(e, c):
            for f in range(fgroups):
                rows[e, pl.ds(f * L, L)] = zv
            return c
        lax.fori_loop(0, BATCH, zbody, 0)
        for k in range(rows_per_tile // BATCH):
            base = row_base + k * BATCH
            pltpu.sync_copy(rows, acc.at[pl.ds(base, BATCH)])
        rem = rows_per_tile % BATCH
        if rem:
            base = row_base + (rows_per_tile // BATCH) * BATCH
            pltpu.sync_copy(rows.at[pl.ds(0, rem)],
                            acc.at[pl.ds(base, rem)])
        plsc.subcore_barrier()

        pltpu.sync_copy(dinv_hbm, dinv_loc)

        def compute_norm(j):
            for g in range(FG):
                sl = pl.ds(g * L, L)
                rv = rowbuf[j, sl]
                cv = colbuf[j, sl]
                wv = wbuf[j, sl]
                dr = plsc.load_gather(dinv_loc, [rv])
                dc = plsc.load_gather(dinv_loc, [cv])
                normbuf[sl] = dr * wv * dc

        def scale_rows(buf):
            def ebody(e, c3):
                ei = jnp.broadcast_to(e, (L,)).astype(jnp.int32)
                s = plsc.load_gather(normbuf, [ei])
                for f in range(fgroups):
                    fs = pl.ds(f * L, L)
                    buf[e, fs] = buf[e, fs] * s
                return c3
            lax.fori_loop(0, BATCH, ebody, 0, unroll=4)

        bufs = (rows_a, rows_b)
        gsems = (gsem_a, gsem_b)
        ssems = (ssem_a, ssem_b)

        def stage_body(st, c):
            chunk = wid * n_stages + st
            pltpu.sync_copy(row_g.at[chunk], rowbuf)
            pltpu.sync_copy(col_g.at[chunk], colbuf)
            pltpu.sync_copy(w_g.at[chunk], wbuf)

            gath = [None, None]
            scat = [None, None]
            gath[0] = pltpu.async_copy(
                h_hbm.at[rowbuf.at[0]], bufs[0], gsems[0])
            for j in range(sb):
                p = j % 2
                q = (j + 1) % 2
                if j + 1 < sb:
                    if scat[q] is not None:
                        scat[q].wait()
                    gath[q] = pltpu.async_copy(
                        h_hbm.at[rowbuf.at[j + 1]], bufs[q], gsems[q])
                compute_norm(j)
                gath[p].wait()
                scale_rows(bufs[p])
                scat[p] = pltpu.async_copy(
                    bufs[p], acc.at[colbuf.at[j]], ssems[p], add=True)
            scat[0].wait()
            scat[1].wait()
            return c
        lax.fori_loop(0, n_stages, stage_body, 0)
        plsc.subcore_barrier()

        src = acc.at[pl.ds(row_base, rows_per_tile)]

        @pl.when(cid == 0)
        def _():
            pltpu.sync_copy(src, out0.at[pl.ds(row_base, rows_per_tile)])

        @pl.when(cid == 1)
        def _():
            pltpu.sync_copy(src, out1.at[pl.ds(row_base, rows_per_tile)])

    return gcn_sc


def _tc_matmul_dinv(x, W, deg0, deg1, n_vec):
    n, d_in = x.shape
    d_out = W.shape[1]
    blk = 400
    rows_dinv = n_vec // 128

    def body(x_ref, w_ref, d0_ref, d1_ref, h_ref, dinv_ref):
        h_ref[...] = jnp.dot(x_ref[...], w_ref[...],
                             preferred_element_type=jnp.float32)
        deg = d0_ref[...] + d1_ref[...]
        dinv_ref[...] = jnp.where(
            deg > 0, lax.rsqrt(jnp.maximum(deg, 1e-12)), 0.0)

    h, dinv = pl.pallas_call(
        body,
        grid=(n // blk,),
        in_specs=[pl.BlockSpec((blk, d_in), lambda i: (i, 0)),
                  pl.BlockSpec((d_in, d_out), lambda i: (0, 0)),
                  pl.BlockSpec((rows_dinv, 128), lambda i: (0, 0)),
                  pl.BlockSpec((rows_dinv, 128), lambda i: (0, 0))],
        out_specs=[pl.BlockSpec((blk, d_out), lambda i: (i, 0)),
                   pl.BlockSpec((rows_dinv, 128), lambda i: (0, 0))],
        out_shape=[jax.ShapeDtypeStruct((n, d_out), jnp.float32),
                   jax.ShapeDtypeStruct((rows_dinv, 128), jnp.float32)],
    )(x, W, deg0.reshape(rows_dinv, 128), deg1.reshape(rows_dinv, 128))
    return h, dinv.reshape(n_vec)


def _tc_combine(p0, p1, b2d, n, d):
    blk = 80
    return pl.pallas_call(
        lambda a_ref, b_ref, c_ref, o_ref: o_ref.__setitem__(
            ..., a_ref[...] + b_ref[...] + c_ref[...]),
        grid=(n // blk,),
        in_specs=[pl.BlockSpec((blk, d), lambda i: (i, 0)),
                  pl.BlockSpec((blk, d), lambda i: (i, 0)),
                  pl.BlockSpec((1, d), lambda i: (0, 0))],
        out_specs=pl.BlockSpec((blk, d), lambda i: (i, 0)),
        out_shape=jax.ShapeDtypeStruct((n, d), jnp.float32),
    )(p0, p1, b2d)


def kernel(x, edge_index, edge_weight, W, b):
    n, d_in = x.shape
    d = W.shape[1]
    e = edge_index.shape[1]

    e_full = e + n
    per_tile_edges = -(-e_full // (NW * BATCH)) * BATCH
    e_pad = per_tile_edges * NW
    nb = per_tile_edges // BATCH
    pad = e_pad - e_full

    idx_dtype = edge_index.dtype
    loop_idx = jnp.arange(n, dtype=idx_dtype)
    zpad_i = jnp.zeros((pad,), dtype=idx_dtype)
    row_full = jnp.concatenate([edge_index[0], loop_idx, zpad_i])
    col_full = jnp.concatenate([edge_index[1], loop_idx, zpad_i])
    w_full = jnp.concatenate([edge_weight, jnp.ones((n,), jnp.float32),
                              jnp.zeros((pad,), jnp.float32)])
    row_g = row_full.reshape(NW, nb, BATCH).astype(jnp.int32)
    col_g = col_full.reshape(NW, nb, BATCH).astype(jnp.int32)
    w_g = w_full.reshape(NW, nb, BATCH)

    n_pad = -(-n // (NS * 8)) * (NS * 8)
    n_vec = -(-n // (NS * L)) * (NS * L)

    sb = next(s for s in (9, 3, 1) if nb % s == 0)
    n_stages = nb // sb
    row_s = row_g.reshape(NW * n_stages, sb, BATCH)
    col_s = col_g.reshape(NW * n_stages, sb, BATCH)
    w_s = w_g.reshape(NW * n_stages, sb, BATCH)
    deg0, deg1 = _sc_deg(n_vec, nb)(col_g, w_g)
    h, dinv = _tc_matmul_dinv(x, W, deg0, deg1, n_vec)
    p0, p1 = _sc_main(n_pad, n_vec, nb, d, sb)(row_s, col_s, w_s, h, dinv)
    out = _tc_combine(p0, p1, b.reshape(1, d), n, d)
    return out
